# Initial kernel scaffold; baseline (speedup 1.0000x reference)
import jax
import jax.numpy as jnp
from jax.experimental import pallas as pl


def kernel(input):
    raise NotImplementedError("write your pallas kernel here")


# speedup vs baseline: 12.9585x; 12.9585x over previous
import jax
import jax.numpy as jnp
from jax import lax
from jax.experimental import pallas as pl
from jax.experimental.pallas import tpu as pltpu
from jax.experimental.pallas import tpu_sc as plsc

_R, _N = 64, 32768
_L = 16
_NCHUNK = _N // _L
_NC, _NS = 2, 16
_NW = _NC * _NS
_ROWS_PER_W = _R // _NW
_NEG = -3.0e38


def _sparsemax_body(x_hbm, out_hbm, row_v, comp_v):
    c = lax.axis_index("c")
    s = lax.axis_index("s")
    wid = s * _NC + c

    for t in range(_ROWS_PER_W):
        r = wid * _ROWS_PER_W + t
        pltpu.sync_copy(x_hbm.at[r], row_v)

        def max_body(i, acc):
            return jnp.maximum(acc, row_v[pl.ds(i * _L, _L)])

        acc = lax.fori_loop(0, _NCHUNK, max_body,
                            jnp.full((_L,), _NEG, jnp.float32))
        mx = jnp.max(acc)
        t0 = mx - 1.0

        def comp_body(i, wp):
            v = row_v[pl.ds(i * _L, _L)]
            m = v > t0
            pos = plsc.cumsum(m.astype(jnp.int32)) - 1 + wp
            plsc.store_scatter(comp_v, [pos], v, mask=m)
            return wp + jnp.sum(m.astype(jnp.int32))

        cnt = lax.fori_loop(0, _NCHUNK, comp_body, jnp.int32(0))
        nch = (cnt + _L - 1) // _L

        def stats_body(i, carry):
            S, K, tau = carry
            v = comp_v[pl.ds(i * _L, _L)]
            valid = (lax.iota(jnp.int32, _L) + i * _L) < cnt
            m = valid & (v > tau)
            return (S + jnp.sum(jnp.where(m, v, 0.0)),
                    K + jnp.sum(jnp.where(m, 1, 0)), tau)

        def mich_cond(carry):
            tau, prev, it = carry
            return (tau > prev) & (it < jnp.int32(64))

        def mich_body(carry):
            tau, prev, it = carry
            S, K, _ = lax.fori_loop(0, nch, stats_body,
                                    (jnp.float32(0.0), jnp.int32(0), tau))
            sv = jnp.full((_L,), S - 1.0, jnp.float32)
            kv = jnp.full((_L,), jnp.maximum(K, 1), jnp.int32).astype(jnp.float32)
            nt = jnp.max(sv / kv)
            return (nt, tau, it + 1)

        tau, _, _ = lax.while_loop(mich_cond, mich_body,
                                   (t0, t0 - 1.0, jnp.int32(0)))

        def out_body(i, carry):
            sl = pl.ds(i * _L, _L)
            row_v[sl] = jnp.maximum(row_v[sl] - tau, 0.0)
            return carry

        lax.fori_loop(0, _NCHUNK, out_body, jnp.int32(0))
        pltpu.sync_copy(row_v, out_hbm.at[r])


def kernel(input):
    f = pl.kernel(
        _sparsemax_body,
        out_type=jax.ShapeDtypeStruct((_R, _N), jnp.float32),
        mesh=plsc.VectorSubcoreMesh(core_axis_name="c", subcore_axis_name="s"),
        compiler_params=pltpu.CompilerParams(needs_layout_passes=False),
        scratch_types=[
            pltpu.VMEM((_N,), jnp.float32),
            pltpu.VMEM((_N + _L,), jnp.float32),
        ],
    )
    return f(input)


# speedup vs baseline: 21.0322x; 1.6230x over previous
import jax
import jax.numpy as jnp
from jax import lax
from jax.experimental import pallas as pl
from jax.experimental.pallas import tpu as pltpu
from jax.experimental.pallas import tpu_sc as plsc

_R, _N = 64, 32768
_L = 16
_NCHUNK = _N // _L
_GRP = 16
_NGRP = _NCHUNK // _GRP
_NC, _NS = 2, 16
_NW = _NC * _NS
_ROWS_PER_W = _R // _NW
_NEG = -3.0e38


def _row_tau(row_v, gmax_v, comp_v):
    def max_body(g, gl):
        base = g * (_GRP * _L)
        acc = row_v[pl.ds(base, _L)]
        for u in range(1, _GRP):
            acc = jnp.maximum(acc, row_v[pl.ds(base + u * _L, _L)])
        gmax_v[pl.ds(g * _L, _L)] = acc
        return jnp.maximum(gl, acc)

    gl = lax.fori_loop(0, _NGRP, max_body,
                       jnp.full((_L,), _NEG, jnp.float32))
    mx = jnp.max(gl)
    t0 = mx - 1.0

    def disc_body(g, wp):
        gv = gmax_v[pl.ds(g * _L, _L)]
        anyc = jnp.sum((gv > t0).astype(jnp.int32))

        def rescan(wp):
            base = g * (_GRP * _L)
            for u in range(_GRP):
                v = row_v[pl.ds(base + u * _L, _L)]
                m = v > t0
                c = jnp.sum(m.astype(jnp.int32))

                def compact(wp):
                    plsc.store_compressed(comp_v.at[pl.ds(wp, _L)], v, mask=m)
                    return wp + c

                wp = lax.cond(c > 0, compact, lambda wp: wp, wp)
            return wp

        return lax.cond(anyc > 0, rescan, lambda wp: wp, wp)

    cnt = lax.fori_loop(0, _NGRP, disc_body, jnp.int32(0))
    nch = (cnt + _L - 1) // _L

    def stats_body(i, carry):
        S, K, tau = carry
        v = comp_v[pl.ds(i * _L, _L)]
        valid = (lax.iota(jnp.int32, _L) + i * _L) < cnt
        m = valid & (v > tau)
        return (S + jnp.sum(jnp.where(m, v, 0.0)),
                K + jnp.sum(jnp.where(m, 1, 0)), tau)

    def mich_cond(carry):
        tau, prev, it = carry
        return (tau > prev) & (it < jnp.int32(64))

    def mich_body(carry):
        tau, prev, it = carry
        S, K, _ = lax.fori_loop(0, nch, stats_body,
                                (jnp.float32(0.0), jnp.int32(0), tau))
        sv = jnp.full((_L,), S - 1.0, jnp.float32)
        kv = jnp.full((_L,), jnp.maximum(K, 1), jnp.int32).astype(jnp.float32)
        nt = jnp.max(sv / kv)
        return (nt, tau, it + 1)

    tau, _, _ = lax.while_loop(mich_cond, mich_body,
                               (t0, t0 - 1.0, jnp.int32(0)))
    return tau


def _emit_output(row_v, tau):
    def out_body(g, carry):
        base = g * (_GRP * _L)
        for u in range(_GRP):
            sl = pl.ds(base + u * _L, _L)
            row_v[sl] = jnp.maximum(row_v[sl] - tau, 0.0)
        return carry

    lax.fori_loop(0, _NGRP, out_body, jnp.int32(0))


def _sparsemax_body(x_hbm, out_hbm, row_a, row_b, gmax_v, comp_v,
                    sem_a, sem_b, sem_out):
    c = lax.axis_index("c")
    s = lax.axis_index("s")
    wid = s * _NC + c
    r0 = wid * _ROWS_PER_W
    r1 = r0 + 1

    cp_a = pltpu.async_copy(x_hbm.at[r0], row_a, sem_a)
    cp_b = pltpu.async_copy(x_hbm.at[r1], row_b, sem_b)

    cp_a.wait()
    tau_a = _row_tau(row_a, gmax_v, comp_v)
    _emit_output(row_a, tau_a)
    out_a = pltpu.async_copy(row_a, out_hbm.at[r0], sem_out)

    cp_b.wait()
    tau_b = _row_tau(row_b, gmax_v, comp_v)
    _emit_output(row_b, tau_b)
    out_a.wait()
    pltpu.async_copy(row_b, out_hbm.at[r1], sem_out).wait()


def kernel(input):
    f = pl.kernel(
        _sparsemax_body,
        out_type=jax.ShapeDtypeStruct((_R, _N), jnp.float32),
        mesh=plsc.VectorSubcoreMesh(core_axis_name="c", subcore_axis_name="s"),
        compiler_params=pltpu.CompilerParams(needs_layout_passes=False),
        scratch_types=[
            pltpu.VMEM((_N,), jnp.float32),
            pltpu.VMEM((_N,), jnp.float32),
            pltpu.VMEM((_NGRP * _L,), jnp.float32),
            pltpu.VMEM((_N + _L,), jnp.float32),
            pltpu.SemaphoreType.DMA,
            pltpu.SemaphoreType.DMA,
            pltpu.SemaphoreType.DMA,
        ],
    )
    return f(input)


# speedup vs baseline: 38.9454x; 1.8517x over previous
import jax
import jax.numpy as jnp
from jax import lax
from jax.experimental import pallas as pl
from jax.experimental.pallas import tpu as pltpu
from jax.experimental.pallas import tpu_sc as plsc

_R, _N = 64, 32768
_L = 16
_NCHUNK = _N // _L
_GRP = 16
_NGRP = _NCHUNK // _GRP
_NCOL = _NGRP * _L
_NC, _NS = 2, 16
_NW = _NC * _NS
_ROWS_PER_W = _R // _NW
_NEG = -3.0e38


def _row_tau(row_v, gmax_v, colid_v, colval_v):
    @plsc.parallel_loop(0, _NGRP, step=1, unroll=2,
                        carry=jnp.full((_L,), _NEG, jnp.float32))
    def max_loop(g, gl):
        base = g * (_GRP * _L)
        vs = [row_v[pl.ds(base + u * _L, _L)] for u in range(_GRP)]
        while len(vs) > 1:
            vs = [jnp.maximum(vs[2 * i], vs[2 * i + 1])
                  for i in range(len(vs) // 2)]
        gmax_v[pl.ds(g * _L, _L)] = vs[0]
        return jnp.maximum(gl, vs[0])

    mx = jnp.max(max_loop)
    t0 = mx - 1.0

    def colcap_body(b, wc):
        gv = gmax_v[pl.ds(b * _L, _L)]
        m = gv > t0
        ids = lax.iota(jnp.int32, _L) + b * _L
        plsc.store_compressed(colid_v.at[pl.ds(wc, _L)], ids, mask=m)
        return wc + plsc.all_reduce_population_count(m)[0]

    wc = lax.fori_loop(0, _NGRP, colcap_body, jnp.int32(0))

    def gather_body(i, carry):
        cid = colid_v[pl.ds(i, _L)][0]
        base = (cid >> 4) * (_GRP * _L) + (cid & (_L - 1))
        idx = base + lax.iota(jnp.int32, _L) * _L
        colval_v[pl.ds(i * _L, _L)] = plsc.load_gather(row_v, [idx])
        return carry

    lax.fori_loop(0, wc, gather_body, jnp.int32(0))

    def mich_cond(carry):
        tau, prev, it = carry
        return (tau > prev) & (it < jnp.int32(64))

    def mich_body(carry):
        tau, prev, it = carry

        def sbody(i, c2):
            sv, kv = c2
            v = colval_v[pl.ds(i * _L, _L)]
            m = v > tau
            return (sv + jnp.where(m, v, 0.0), kv + jnp.where(m, 1, 0))

        sv, kv = lax.fori_loop(
            0, wc, sbody,
            (jnp.zeros((_L,), jnp.float32), jnp.zeros((_L,), jnp.int32)))
        S = jnp.sum(sv)
        K = jnp.sum(kv)
        num = jnp.full((_L,), S - 1.0, jnp.float32)
        den = jnp.full((_L,), jnp.maximum(K, 1), jnp.int32).astype(jnp.float32)
        nt = jnp.max(num / den)
        return (nt, tau, it + 1)

    tau, _, _ = lax.while_loop(mich_cond, mich_body,
                               (t0, t0 - 1.0, jnp.int32(0)))
    return tau


def _emit_output(row_v, tau):
    @plsc.parallel_loop(0, _NGRP, step=1, unroll=2)
    def out_loop(g):
        base = g * (_GRP * _L)
        for u in range(_GRP):
            sl = pl.ds(base + u * _L, _L)
            row_v[sl] = jnp.maximum(row_v[sl] - tau, 0.0)


def _sparsemax_body(x_hbm, out_hbm, row_a, row_b, gmax_v, colid_v, colval_v,
                    sem_a, sem_b, sem_out):
    c = lax.axis_index("c")
    s = lax.axis_index("s")
    wid = s * _NC + c
    r0 = wid * _ROWS_PER_W
    r1 = r0 + 1

    cp_a = pltpu.async_copy(x_hbm.at[r0], row_a, sem_a)
    cp_b = pltpu.async_copy(x_hbm.at[r1], row_b, sem_b)

    cp_a.wait()
    tau_a = _row_tau(row_a, gmax_v, colid_v, colval_v)
    _emit_output(row_a, tau_a)
    out_a = pltpu.async_copy(row_a, out_hbm.at[r0], sem_out)

    cp_b.wait()
    tau_b = _row_tau(row_b, gmax_v, colid_v, colval_v)
    _emit_output(row_b, tau_b)
    out_a.wait()
    pltpu.async_copy(row_b, out_hbm.at[r1], sem_out).wait()


def kernel(input):
    f = pl.kernel(
        _sparsemax_body,
        out_type=jax.ShapeDtypeStruct((_R, _N), jnp.float32),
        mesh=plsc.VectorSubcoreMesh(core_axis_name="c", subcore_axis_name="s"),
        compiler_params=pltpu.CompilerParams(needs_layout_passes=False),
        scratch_types=[
            pltpu.VMEM((_N,), jnp.float32),
            pltpu.VMEM((_N,), jnp.float32),
            pltpu.VMEM((_NCOL,), jnp.float32),
            pltpu.VMEM((_NCOL + _L,), jnp.int32),
            pltpu.VMEM((_N,), jnp.float32),
            pltpu.SemaphoreType.DMA,
            pltpu.SemaphoreType.DMA,
            pltpu.SemaphoreType.DMA,
        ],
    )
    return f(input)


# speedup vs baseline: 41.7044x; 1.0708x over previous
import jax
import jax.numpy as jnp
from jax import lax
from jax.experimental import pallas as pl
from jax.experimental.pallas import tpu as pltpu
from jax.experimental.pallas import tpu_sc as plsc

_R, _N = 64, 32768
_L = 16
_NCHUNK = _N // _L
_GRP = 16
_NGRP = _NCHUNK // _GRP
_NCOL = _NGRP * _L
_NBLK = _NGRP // _L
_NC, _NS = 2, 16
_NW = _NC * _NS
_ROWS_PER_W = _R // _NW
_NQ = 4
_QW = _N // _NQ
_QGRP = _NGRP // _NQ
_NEG = -3.0e38


def _max_quarter(row_v, gmax_v, q, gl):
    @plsc.parallel_loop(q * _QGRP, (q + 1) * _QGRP, step=1, unroll=2,
                        carry=gl)
    def max_loop(g, acc):
        base = g * (_GRP * _L)
        vs = [row_v[pl.ds(base + u * _L, _L)] for u in range(_GRP)]
        while len(vs) > 1:
            vs = [jnp.maximum(vs[2 * i], vs[2 * i + 1])
                  for i in range(len(vs) // 2)]
        gmax_v[pl.ds(g * _L, _L)] = vs[0]
        return jnp.maximum(acc, vs[0])

    return max_loop


def _row_tau(row_v, gmax_v, cnts_v, offs_v, colid_v, colval_v, gl):
    mx = jnp.max(gl)
    t0 = mx - 1.0
    lane0 = lax.iota(jnp.int32, _L) == 0

    @plsc.parallel_loop(0, _NGRP, step=1, unroll=2)
    def cnt_loop(g):
        gv = gmax_v[pl.ds(g * _L, _L)]
        m = gv > t0
        cnt = plsc.all_reduce_population_count(m)
        plsc.store_scatter(cnts_v, [jnp.full((_L,), g, jnp.int32)], cnt,
                           mask=lane0)

    def scan_body(b, tot):
        c = cnts_v[pl.ds(b * _L, _L)]
        inc = plsc.cumsum(c)
        offs_v[pl.ds(b * _L, _L)] = inc - c + tot
        return tot + inc[_L - 1]

    wc = lax.fori_loop(0, _NBLK, scan_body, jnp.int32(0))

    @plsc.parallel_loop(0, _NGRP, step=1, unroll=2)
    def colcap_loop(g):
        gv = gmax_v[pl.ds(g * _L, _L)]
        m = gv > t0
        off = offs_v[pl.ds(g, _L)][0]
        ids = lax.iota(jnp.int32, _L) + g * _L
        plsc.store_compressed(colid_v.at[pl.ds(off, _L)], ids, mask=m)

    @plsc.parallel_loop(0, wc, step=1, unroll=2)
    def gather_loop(i):
        cid = colid_v[pl.ds(i, _L)][0]
        base = (cid >> 4) * (_GRP * _L) + (cid & (_L - 1))
        idx = base + lax.iota(jnp.int32, _L) * _L
        colval_v[pl.ds(i * _L, _L)] = plsc.load_gather(row_v, [idx])

    def mich_cond(carry):
        tau, prev, it = carry
        return (tau > prev) & (it < jnp.int32(64))

    def mich_body(carry):
        tau, prev, it = carry

        def sbody(i, c2):
            sv, kv = c2
            v = colval_v[pl.ds(i * _L, _L)]
            m = v > tau
            return (sv + jnp.where(m, v, 0.0), kv + jnp.where(m, 1, 0))

        sv, kv = lax.fori_loop(
            0, wc, sbody,
            (jnp.zeros((_L,), jnp.float32), jnp.zeros((_L,), jnp.int32)))
        S = jnp.sum(sv)
        K = jnp.sum(kv)
        num = jnp.full((_L,), S - 1.0, jnp.float32)
        den = jnp.full((_L,), jnp.maximum(K, 1), jnp.int32).astype(jnp.float32)
        nt = jnp.max(num / den)
        return (nt, tau, it + 1)

    tau, _, _ = lax.while_loop(mich_cond, mich_body,
                               (t0, t0 - 1.0, jnp.int32(0)))
    return tau


def _emit_output(row_v, out_row_hbm, tau, sem):
    copies = []
    for q in range(_NQ):

        @plsc.parallel_loop(q * _QGRP, (q + 1) * _QGRP, step=1, unroll=2)
        def out_loop(g):
            base = g * (_GRP * _L)
            for u in range(_GRP):
                sl = pl.ds(base + u * _L, _L)
                row_v[sl] = jnp.maximum(row_v[sl] - tau, 0.0)

        copies.append(pltpu.async_copy(
            row_v.at[pl.ds(q * _QW, _QW)],
            out_row_hbm.at[pl.ds(q * _QW, _QW)], sem))
    return copies


def _sparsemax_body(x_hbm, out_hbm, row_a, row_b, gmax_v, cnts_v, offs_v,
                    colid_v, colval_v, sem_a, sem_b, sem_oa, sem_ob):
    c = lax.axis_index("c")
    s = lax.axis_index("s")
    wid = s * _NC + c
    r0 = wid * _ROWS_PER_W
    r1 = r0 + 1

    cps_a = [pltpu.async_copy(x_hbm.at[r0, pl.ds(q * _QW, _QW)],
                              row_a.at[pl.ds(q * _QW, _QW)], sem_a)
             for q in range(_NQ)]
    cp_b = pltpu.async_copy(x_hbm.at[r1], row_b, sem_b)

    gl = jnp.full((_L,), _NEG, jnp.float32)
    for q in range(_NQ):
        cps_a[q].wait()
        gl = _max_quarter(row_a, gmax_v, q, gl)
    tau_a = _row_tau(row_a, gmax_v, cnts_v, offs_v, colid_v, colval_v, gl)
    out_a = _emit_output(row_a, out_hbm.at[r0], tau_a, sem_oa)

    cp_b.wait()
    gl = jnp.full((_L,), _NEG, jnp.float32)
    for q in range(_NQ):
        gl = _max_quarter(row_b, gmax_v, q, gl)
    tau_b = _row_tau(row_b, gmax_v, cnts_v, offs_v, colid_v, colval_v, gl)
    out_b = _emit_output(row_b, out_hbm.at[r1], tau_b, sem_ob)

    for cp in out_a + out_b:
        cp.wait()


def kernel(input):
    f = pl.kernel(
        _sparsemax_body,
        out_type=jax.ShapeDtypeStruct((_R, _N), jnp.float32),
        mesh=plsc.VectorSubcoreMesh(core_axis_name="c", subcore_axis_name="s"),
        compiler_params=pltpu.CompilerParams(needs_layout_passes=False),
        scratch_types=[
            pltpu.VMEM((_N,), jnp.float32),
            pltpu.VMEM((_N,), jnp.float32),
            pltpu.VMEM((_NCOL,), jnp.float32),
            pltpu.VMEM((_NGRP,), jnp.int32),
            pltpu.VMEM((_NGRP + _L,), jnp.int32),
            pltpu.VMEM((_NCOL + _L,), jnp.int32),
            pltpu.VMEM((_N,), jnp.float32),
            pltpu.SemaphoreType.DMA,
            pltpu.SemaphoreType.DMA,
            pltpu.SemaphoreType.DMA,
            pltpu.SemaphoreType.DMA,
        ],
    )
    return f(input)


# speedup vs baseline: 42.5101x; 1.0193x over previous
import jax
import jax.numpy as jnp
from jax import lax
from jax.experimental import pallas as pl
from jax.experimental.pallas import tpu as pltpu
from jax.experimental.pallas import tpu_sc as plsc

_R, _N = 64, 32768
_L = 16
_NCHUNK = _N // _L
_GRP = 16
_NGRP = _NCHUNK // _GRP
_NCOL = _NGRP * _L
_NBLK = _NGRP // _L
_NC, _NS = 2, 16
_NW = _NC * _NS
_ROWS_PER_W = _R // _NW
_NQ = 4
_QW = _N // _NQ
_QGRP = _NGRP // _NQ
_NEG = -3.0e38


def _max_quarter(row_v, gmax_v, q, gl):
    @plsc.parallel_loop(q * _QGRP, (q + 1) * _QGRP, step=1, unroll=2,
                        carry=gl)
    def max_loop(g, acc):
        base = g * (_GRP * _L)
        vs = [row_v[pl.ds(base + u * _L, _L)] for u in range(_GRP)]
        while len(vs) > 1:
            vs = [jnp.maximum(vs[2 * i], vs[2 * i + 1])
                  for i in range(len(vs) // 2)]
        gmax_v[pl.ds(g * _L, _L)] = vs[0]
        return jnp.maximum(acc, vs[0])

    return max_loop


def _row_tau(row_v, gmax_v, cnts_v, offs_v, colid_v, colval_v, gl):
    mx = jnp.max(gl)
    t0 = mx - 1.0
    lane0 = lax.iota(jnp.int32, _L) == 0

    @plsc.parallel_loop(0, _NGRP, step=1, unroll=2)
    def cnt_loop(g):
        gv = gmax_v[pl.ds(g * _L, _L)]
        m = gv > t0
        cnt = plsc.all_reduce_population_count(m)
        plsc.store_scatter(cnts_v, [jnp.full((_L,), g, jnp.int32)], cnt,
                           mask=lane0)

    def scan_body(b, tot):
        c = cnts_v[pl.ds(b * _L, _L)]
        inc = plsc.cumsum(c)
        offs_v[pl.ds(b * _L, _L)] = inc - c + tot
        return tot + inc[_L - 1]

    wc = lax.fori_loop(0, _NBLK, scan_body, jnp.int32(0))

    @plsc.parallel_loop(0, _NGRP, step=1, unroll=2)
    def colcap_loop(g):
        gv = gmax_v[pl.ds(g * _L, _L)]
        m = gv > t0
        off = offs_v[pl.ds(g, _L)][0]
        ids = lax.iota(jnp.int32, _L) + g * _L
        plsc.store_compressed(colid_v.at[pl.ds(off, _L)], ids, mask=m)

    @plsc.parallel_loop(0, wc, step=1, unroll=2)
    def gather_loop(i):
        cid = colid_v[pl.ds(i, _L)][0]
        base = (cid >> 4) * (_GRP * _L) + (cid & (_L - 1))
        idx = base + lax.iota(jnp.int32, _L) * _L
        colval_v[pl.ds(i * _L, _L)] = plsc.load_gather(row_v, [idx])

    def mich_cond(carry):
        tau, prev, it = carry
        return (tau > prev) & (it < jnp.int32(64))

    def mich_body(carry):
        tau, prev, it = carry

        def sbody(i, c2):
            sv, kv = c2
            v = colval_v[pl.ds(i * _L, _L)]
            m = v > tau
            return (sv + jnp.where(m, v, 0.0), kv + jnp.where(m, 1, 0))

        sv, kv = lax.fori_loop(
            0, wc, sbody,
            (jnp.zeros((_L,), jnp.float32), jnp.zeros((_L,), jnp.int32)))
        S = jnp.sum(sv)
        K = jnp.sum(kv)
        num = jnp.full((_L,), S - 1.0, jnp.float32)
        den = jnp.full((_L,), jnp.maximum(K, 1), jnp.int32).astype(jnp.float32)
        nt = jnp.max(num / den)
        return (nt, tau, it + 1)

    tau, _, _ = lax.while_loop(mich_cond, mich_body,
                               (t0, t0 - 1.0, jnp.int32(0)))
    return tau


def _emit_output(row_v, out_row_hbm, tau, sem):
    copies = []
    for q in range(_NQ):

        @plsc.parallel_loop(q * _QGRP, (q + 1) * _QGRP, step=1, unroll=2)
        def out_loop(g):
            base = g * (_GRP * _L)
            for u in range(_GRP):
                sl = pl.ds(base + u * _L, _L)
                row_v[sl] = jnp.maximum(row_v[sl] - tau, 0.0)

        copies.append(pltpu.async_copy(
            row_v.at[pl.ds(q * _QW, _QW)],
            out_row_hbm.at[pl.ds(q * _QW, _QW)], sem))
    return copies


def _sparsemax_body(x_hbm, out_hbm, row_a, row_b, gmax_v, cnts_v, offs_v,
                    colid_v, colval_v, sem_a, sem_b, sem_oa, sem_ob):
    c = lax.axis_index("c")
    s = lax.axis_index("s")
    wid = s * _NC + c
    r0 = wid * _ROWS_PER_W
    r1 = r0 + 1

    cps_a = [pltpu.async_copy(x_hbm.at[r0, pl.ds(q * _QW, _QW)],
                              row_a.at[pl.ds(q * _QW, _QW)], sem_a)
             for q in range(_NQ)]

    gl = jnp.full((_L,), _NEG, jnp.float32)
    cp_b = None
    for q in range(_NQ):
        cps_a[q].wait()
        if q == 0:
            cp_b = pltpu.async_copy(x_hbm.at[r1], row_b, sem_b)
        gl = _max_quarter(row_a, gmax_v, q, gl)
    tau_a = _row_tau(row_a, gmax_v, cnts_v, offs_v, colid_v, colval_v, gl)
    out_a = _emit_output(row_a, out_hbm.at[r0], tau_a, sem_oa)

    cp_b.wait()
    gl = jnp.full((_L,), _NEG, jnp.float32)
    for q in range(_NQ):
        gl = _max_quarter(row_b, gmax_v, q, gl)
    tau_b = _row_tau(row_b, gmax_v, cnts_v, offs_v, colid_v, colval_v, gl)
    out_b = _emit_output(row_b, out_hbm.at[r1], tau_b, sem_ob)

    for cp in out_a + out_b:
        cp.wait()


def kernel(input):
    f = pl.kernel(
        _sparsemax_body,
        out_type=jax.ShapeDtypeStruct((_R, _N), jnp.float32),
        mesh=plsc.VectorSubcoreMesh(core_axis_name="c", subcore_axis_name="s"),
        compiler_params=pltpu.CompilerParams(needs_layout_passes=False,
                                             skip_device_barrier=True),
        scratch_types=[
            pltpu.VMEM((_N,), jnp.float32),
            pltpu.VMEM((_N,), jnp.float32),
            pltpu.VMEM((_NCOL,), jnp.float32),
            pltpu.VMEM((_NGRP,), jnp.int32),
            pltpu.VMEM((_NGRP + _L,), jnp.int32),
            pltpu.VMEM((_NCOL + _L,), jnp.int32),
            pltpu.VMEM((_N,), jnp.float32),
            pltpu.SemaphoreType.DMA,
            pltpu.SemaphoreType.DMA,
            pltpu.SemaphoreType.DMA,
            pltpu.SemaphoreType.DMA,
        ],
    )
    return f(input)
